# Initial kernel scaffold; baseline (speedup 1.0000x reference)
import jax
import jax.numpy as jnp
from jax.experimental import pallas as pl


def kernel(x, edge_index, weight, bias):
    raise NotImplementedError("write your pallas kernel here")


# speedup vs baseline: 3.1401x; 3.1401x over previous
import functools

import jax
import jax.numpy as jnp
from jax import lax
from jax.experimental import pallas as pl
from jax.experimental.pallas import tpu as pltpu
from jax.experimental.pallas import tpu_sc as plsc

N_NODES = 10000
N_EDGES = 320000
D = 128

NC = 2
NS = 16
NW = NC * NS
C = 128
CPT = 80
E_PAD = NW * C * CPT
N_ACC = 10112
ROWS_TILE = N_ACC // NS


def _sc_aggregate(x, src2d, dst2d, zeros_hbm):
    mesh = plsc.VectorSubcoreMesh(core_axis_name="c", subcore_axis_name="s")

    @functools.partial(
        pl.kernel,
        out_type=jax.ShapeDtypeStruct((NC, N_ACC, D), jnp.float32),
        mesh=mesh,
        scratch_types=[
            pltpu.VMEM((CPT, C), jnp.int32),
            pltpu.VMEM((CPT, C), jnp.int32),
            pltpu.VMEM((C, D), jnp.float32),
            pltpu.VMEM_SHARED((N_ACC, D), jnp.float32),
            pltpu.SemaphoreType.DMA,
        ],
    )
    def agg(x_hbm, src_hbm, dst_hbm, zero_hbm, out_hbm,
            src_v, dst_v, rows_v, acc, sem):
        c = lax.axis_index("c")
        s = lax.axis_index("s")
        pltpu.sync_copy(zero_hbm.at[pl.ds(s * ROWS_TILE, ROWS_TILE)],
                        acc.at[pl.ds(s * ROWS_TILE, ROWS_TILE)])
        base = (c * NS + s) * CPT
        pltpu.sync_copy(src_hbm.at[pl.ds(base, CPT)], src_v)
        pltpu.sync_copy(dst_hbm.at[pl.ds(base, CPT)], dst_v)
        plsc.subcore_barrier()

        def step(i, carry):
            pltpu.async_copy(x_hbm.at[src_v.at[i]], rows_v, sem).wait()
            pltpu.sync_copy(rows_v, acc.at[dst_v.at[i]], add=True)
            return carry

        lax.fori_loop(0, CPT, step, 0)
        plsc.subcore_barrier()
        pltpu.sync_copy(acc.at[pl.ds(s * ROWS_TILE, ROWS_TILE)],
                        out_hbm.at[c, pl.ds(s * ROWS_TILE, ROWS_TILE)])

    return agg(x, src2d, dst2d, zeros_hbm)


def _tc_matmul(p0, p1, weight, bias2d):
    blk = 1000

    def body(p0_ref, p1_ref, w_ref, b_ref, o_ref):
        agg = p0_ref[...] + p1_ref[...]
        y = jnp.dot(agg, w_ref[...], preferred_element_type=jnp.float32)
        o_ref[...] = jnp.maximum(y + b_ref[...], 0.0)

    return pl.pallas_call(
        body,
        grid=(N_NODES // blk,),
        in_specs=[
            pl.BlockSpec((blk, D), lambda i: (i, 0)),
            pl.BlockSpec((blk, D), lambda i: (i, 0)),
            pl.BlockSpec((D, D), lambda i: (0, 0)),
            pl.BlockSpec((1, D), lambda i: (0, 0)),
        ],
        out_specs=pl.BlockSpec((blk, D), lambda i: (i, 0)),
        out_shape=jax.ShapeDtypeStruct((N_NODES, D), jnp.float32),
    )(p0, p1, weight, bias2d)


def kernel(x, edge_index, weight, bias):
    dst = edge_index[0].astype(jnp.int32)
    src = edge_index[1].astype(jnp.int32)
    pad = E_PAD - N_EDGES
    src2d = jnp.concatenate(
        [src, jnp.zeros((pad,), jnp.int32)]).reshape(NW * CPT, C)
    dst2d = jnp.concatenate(
        [dst, jnp.full((pad,), N_NODES, jnp.int32)]).reshape(NW * CPT, C)
    zeros_hbm = jnp.zeros((N_ACC, D), jnp.float32)
    partials = _sc_aggregate(x, src2d, dst2d, zeros_hbm)
    return _tc_matmul(partials[0], partials[1], weight,
                      bias.reshape(1, D))


# speedup vs baseline: 3.5014x; 1.1151x over previous
import functools

import jax
import jax.numpy as jnp
from jax import lax
from jax.experimental import pallas as pl
from jax.experimental.pallas import tpu as pltpu
from jax.experimental.pallas import tpu_sc as plsc

N_NODES = 10000
N_EDGES = 320000
D = 128

NC = 2
NS = 16
NW = NC * NS
C = 128
CPT = 80
E_PAD = NW * C * CPT
N_ACC = 10112
ROWS_TILE = N_ACC // NS


def _sc_aggregate(x, src2d, dst2d, zeros_hbm):
    mesh = plsc.VectorSubcoreMesh(core_axis_name="c", subcore_axis_name="s")

    @functools.partial(
        pl.kernel,
        out_type=jax.ShapeDtypeStruct((NC, N_ACC, D), jnp.float32),
        mesh=mesh,
        scratch_types=[
            pltpu.VMEM((CPT, C), jnp.int32),
            pltpu.VMEM((1, C), jnp.int32),
            pltpu.VMEM((1, C), jnp.int32),
            pltpu.VMEM((C, D), jnp.float32),
            pltpu.VMEM((C, D), jnp.float32),
            pltpu.VMEM_SHARED((N_ACC, D), jnp.float32),
            pltpu.SemaphoreType.DMA,
            pltpu.SemaphoreType.DMA,
            pltpu.SemaphoreType.DMA,
            pltpu.SemaphoreType.DMA,
        ],
    )
    def agg(x_hbm, src_hbm, dst_hbm, zero_hbm, out_hbm,
            src_v, dst_a, dst_b, rows_a, rows_b, acc,
            sem_ra, sem_rb, sem_da, sem_db):
        c = lax.axis_index("c")
        s = lax.axis_index("s")
        pltpu.sync_copy(zero_hbm.at[pl.ds(s * ROWS_TILE, ROWS_TILE)],
                        acc.at[pl.ds(s * ROWS_TILE, ROWS_TILE)])
        base = (c * NS + s) * CPT
        pltpu.sync_copy(src_hbm.at[pl.ds(base, CPT)], src_v)
        plsc.subcore_barrier()

        pltpu.async_copy(dst_hbm.at[base + 0], dst_a, sem_da)
        pltpu.async_copy(dst_hbm.at[base + 1], dst_b, sem_db)
        pltpu.async_copy(x_hbm.at[src_v.at[0]], rows_a, sem_ra)

        def pair(i, carry):
            a_idx = 2 * i
            b_idx = 2 * i + 1
            pltpu.async_copy(x_hbm.at[src_v.at[b_idx]], rows_b, sem_rb)
            pltpu.make_async_copy(x_hbm.at[src_v.at[a_idx]],
                                  rows_a, sem_ra).wait()
            pltpu.make_async_copy(dst_hbm.at[base + a_idx],
                                  dst_a, sem_da).wait()
            pltpu.sync_copy(rows_a, acc.at[dst_a.at[0]], add=True)

            @pl.when(a_idx + 2 < CPT)
            def _():
                pltpu.async_copy(dst_hbm.at[base + a_idx + 2],
                                 dst_a, sem_da)
                pltpu.async_copy(x_hbm.at[src_v.at[a_idx + 2]],
                                 rows_a, sem_ra)

            pltpu.make_async_copy(x_hbm.at[src_v.at[b_idx]],
                                  rows_b, sem_rb).wait()
            pltpu.make_async_copy(dst_hbm.at[base + b_idx],
                                  dst_b, sem_db).wait()
            pltpu.sync_copy(rows_b, acc.at[dst_b.at[0]], add=True)

            @pl.when(b_idx + 2 < CPT)
            def _():
                pltpu.async_copy(dst_hbm.at[base + b_idx + 2],
                                 dst_b, sem_db)

            return carry

        lax.fori_loop(0, CPT // 2, pair, 0)
        plsc.subcore_barrier()
        pltpu.sync_copy(acc.at[pl.ds(s * ROWS_TILE, ROWS_TILE)],
                        out_hbm.at[c, pl.ds(s * ROWS_TILE, ROWS_TILE)])

    return agg(x, src2d, dst2d, zeros_hbm)


def _tc_matmul(p0, p1, weight, bias2d):
    blk = 1000

    def body(p0_ref, p1_ref, w_ref, b_ref, o_ref):
        agg = p0_ref[...] + p1_ref[...]
        y = jnp.dot(agg, w_ref[...], preferred_element_type=jnp.float32)
        o_ref[...] = jnp.maximum(y + b_ref[...], 0.0)

    return pl.pallas_call(
        body,
        grid=(N_NODES // blk,),
        in_specs=[
            pl.BlockSpec((blk, D), lambda i: (i, 0)),
            pl.BlockSpec((blk, D), lambda i: (i, 0)),
            pl.BlockSpec((D, D), lambda i: (0, 0)),
            pl.BlockSpec((1, D), lambda i: (0, 0)),
        ],
        out_specs=pl.BlockSpec((blk, D), lambda i: (i, 0)),
        out_shape=jax.ShapeDtypeStruct((N_NODES, D), jnp.float32),
    )(p0, p1, weight, bias2d)


def kernel(x, edge_index, weight, bias):
    dst = edge_index[0].astype(jnp.int32)
    src = edge_index[1].astype(jnp.int32)
    pad = E_PAD - N_EDGES
    src2d = jnp.concatenate(
        [src, jnp.zeros((pad,), jnp.int32)]).reshape(NW * CPT, C)
    dst3d = jnp.concatenate(
        [dst, jnp.full((pad,), N_NODES, jnp.int32)]).reshape(NW * CPT, 1, C)
    zeros_hbm = jnp.zeros((N_ACC, D), jnp.float32)
    partials = _sc_aggregate(x, src2d, dst3d, zeros_hbm)
    return _tc_matmul(partials[0], partials[1], weight,
                      bias.reshape(1, D))
